# baseline (speedup 1.0000x reference)
import jax
import jax.numpy as jnp
from jax.experimental import pallas as pl
from jax.experimental.pallas import tpu as pltpu
from jax.experimental.pallas import tpu_sc as plsc

_WINDOW = 128


def _sc_gather(table, idx1d, n, d):
    mesh = plsc.VectorSubcoreMesh(core_axis_name="core", subcore_axis_name="subcore")

    @pl.kernel(
        out_type=jax.ShapeDtypeStruct((n, d), jnp.float32),
        mesh=mesh,
        compiler_params=pltpu.CompilerParams(use_tc_tiling_on_sc=False),
    )
    def gather_kernel(tab_hbm, i_hbm, o_hbm):
        def body(i_vmem, o_vmem):
            pltpu.sync_copy(tab_hbm.at[i_vmem], o_vmem)

        pltpu.emit_pipeline(
            body,
            grid=(n // _WINDOW,),
            in_specs=[pl.BlockSpec((_WINDOW,), index_map=lambda i: (i,))],
            out_specs=[pl.BlockSpec((_WINDOW, d), index_map=lambda i: (i, 0))],
            core_axis_name=("core", "subcore"),
            dimension_semantics=(pltpu.PARALLEL,),
        )(i_hbm, o_hbm)

    return gather_kernel(table, idx1d)


def _tc_layernorm_t(embt, gamma, beta, b, l, d):
    def ln_body(e_ref, g_ref, b_ref, o_ref):
        e = e_ref[...]
        inv = 1.0 / d
        mean = jnp.sum(e, axis=0, keepdims=True) * inv
        msq = jnp.sum(e * e, axis=0, keepdims=True) * inv
        var = msq - mean * mean
        o_ref[0] = (e - mean) * jax.lax.rsqrt(var + 1e-5) * g_ref[...] + b_ref[...]

    return pl.pallas_call(
        ln_body,
        grid=(l,),
        in_specs=[
            pl.BlockSpec((d, b), lambda i: (0, i)),
            pl.BlockSpec((d, 1), lambda i: (0, 0)),
            pl.BlockSpec((d, 1), lambda i: (0, 0)),
        ],
        out_specs=pl.BlockSpec((1, d, b), lambda i: (i, 0, 0)),
        out_shape=jax.ShapeDtypeStruct((l, d, b), jnp.float32),
    )(embt, gamma.reshape(d, 1), beta.reshape(d, 1))


def kernel(x, table, gamma, beta):
    b, l = x.shape
    v, d = table.shape
    n = b * l
    idx1d = x.T.reshape(n).astype(jnp.int32)
    emb = _sc_gather(table, idx1d, n, d)
    embt = emb.T
    out_t = _tc_layernorm_t(embt, gamma, beta, b, l, d)
    return out_t.transpose(2, 0, 1)
